# Initial kernel scaffold; baseline (speedup 1.0000x reference)
import jax
import jax.numpy as jnp
from jax.experimental import pallas as pl


def kernel(x, edge_index, batch, W1e, b1e, gam, bet, W2e, b2e, w_pool, W1b, b1b, W2b, b2b, W1d, b1d, W2d, b2d, Wo, bo):
    raise NotImplementedError("write your pallas kernel here")


# speedup vs baseline: 16.2013x; 16.2013x over previous
import functools
import math

import jax
import jax.numpy as jnp
from jax import lax
from jax.experimental import pallas as pl
from jax.experimental.pallas import tpu as pltpu
from jax.experimental.pallas import tpu_sc as plsc

_BLK = 1024
_CHUNK = 128
_NSUB = 16
_NCORE = 2
_ZROWS = 64


def _elu(v):
    return jnp.where(v > 0, v, jnp.exp(jnp.minimum(v, 0.0)) - 1.0)


def _mm(a, w):
    return jnp.dot(a.astype(jnp.bfloat16), w.astype(jnp.bfloat16),
                   preferred_element_type=jnp.float32)


def _enc_body(x_ref, a0_ref, a1_ref, b1_ref, g_ref, be_ref, w1_ref, w2_ref,
              b2_ref, wp_ref, x1_ref, s_ref):
    h = x_ref[...] + a0_ref[...] + a1_ref[...]
    z = _mm(h, w1_ref[...]) + b1_ref[...]
    z = g_ref[...] * (z * (1.0 / math.sqrt(1.0 + 1e-5))) + be_ref[...]
    x1 = _elu(_mm(_elu(z), w2_ref[...]) + b2_ref[...])
    x1_ref[...] = x1
    wp = wp_ref[...]
    inv = 1.0 / (jnp.sqrt(jnp.sum(wp * wp)) + 1e-16)
    s_ref[...] = _mm(x1, wp.reshape(-1, 1)) * inv


def _encoder(x_pad, a0, a1, b1, g, be, W1, W2, b2, wp):
    NP, D = x_pad.shape
    H = W1.shape[1]
    big = pl.BlockSpec((_BLK, D), lambda i: (i, 0))
    mat = pl.BlockSpec((_BLK, H), lambda i: (i, 0))
    vec = lambda: pl.BlockSpec((1, H), lambda i: (0, 0))
    return pl.pallas_call(
        _enc_body,
        grid=(NP // _BLK,),
        in_specs=[big, big, big, vec(), vec(), vec(),
                  pl.BlockSpec((D, H), lambda i: (0, 0)),
                  pl.BlockSpec((H, H), lambda i: (0, 0)), vec(), vec()],
        out_specs=[mat, pl.BlockSpec((_BLK, 1), lambda i: (i, 0))],
        out_shape=[jax.ShapeDtypeStruct((NP, H), jnp.float32),
                   jax.ShapeDtypeStruct((NP, 1), jnp.float32)],
    )(x_pad, a0, a1, b1, g, be, W1, W2, b2, wp)


def _thresh_body(s_ref, t_ref, *, n_valid, k):
    s = s_ref[...]
    u = lax.bitcast_convert_type(s, jnp.uint32)
    key = jnp.where(u >= jnp.uint32(0x80000000), ~u,
                    u | jnp.uint32(0x80000000))
    rows, cols = s.shape
    idx = (lax.broadcasted_iota(jnp.int32, (rows, cols), 0) * cols
           + lax.broadcasted_iota(jnp.int32, (rows, cols), 1))
    key = jnp.where(idx < n_valid, key, jnp.uint32(0))

    def body(i, t):
        shift = (jnp.int32(31) - i).astype(jnp.uint32)
        cand = t | jnp.left_shift(jnp.uint32(1), shift)
        cnt = jnp.sum((key >= cand).astype(jnp.int32))
        return jnp.where(cnt >= k, cand, t)

    t = lax.fori_loop(0, 32, body, jnp.uint32(0))
    u_orig = jnp.where(t >= jnp.uint32(0x80000000),
                       t ^ jnp.uint32(0x80000000), ~t)
    t_ref[0, 0] = lax.bitcast_convert_type(u_orig, jnp.float32)


def _threshold(score2d, n_valid, k):
    R, C = score2d.shape
    return pl.pallas_call(
        functools.partial(_thresh_body, n_valid=n_valid, k=k),
        in_specs=[pl.BlockSpec((R, C), lambda: (0, 0))],
        out_specs=pl.BlockSpec(memory_space=pltpu.SMEM),
        out_shape=jax.ShapeDtypeStruct((1, 1), jnp.float32),
    )(score2d)


def _pool_body(x1_ref, s_ref, t_ref, xp_ref):
    t = t_ref[0, 0]
    s = s_ref[...]
    xp_ref[...] = jnp.where(s >= t, x1_ref[...] * jnp.tanh(s), 0.0)


def _pool(x1, score, t):
    NP, H = x1.shape
    mat = pl.BlockSpec((_BLK, H), lambda i: (i, 0))
    return pl.pallas_call(
        _pool_body,
        grid=(NP // _BLK,),
        in_specs=[mat, pl.BlockSpec((_BLK, 1), lambda i: (i, 0)),
                  pl.BlockSpec(memory_space=pltpu.SMEM)],
        out_specs=mat,
        out_shape=jax.ShapeDtypeStruct((NP, H), jnp.float32),
    )(x1, score, t)


def _bott_body(xp_ref, a0_ref, a1_ref, b1_ref, w1_ref, w2_ref, b2_ref, s_ref,
               t_ref, xb_ref):
    t = t_ref[0, 0]
    h = xp_ref[...] + a0_ref[...] + a1_ref[...]
    xb = _elu(_mm(_elu(_mm(h, w1_ref[...]) + b1_ref[...]), w2_ref[...])
              + b2_ref[...])
    xb_ref[...] = jnp.where(s_ref[...] >= t, xb, 0.0)


def _bottleneck(xp, a0, a1, b1, W1, W2, b2, score, t):
    NP, H = xp.shape
    mat = pl.BlockSpec((_BLK, H), lambda i: (i, 0))
    vec = lambda: pl.BlockSpec((1, H), lambda i: (0, 0))
    sq = lambda: pl.BlockSpec((H, H), lambda i: (0, 0))
    return pl.pallas_call(
        _bott_body,
        grid=(NP // _BLK,),
        in_specs=[mat, mat, mat, vec(), sq(), sq(), vec(),
                  pl.BlockSpec((_BLK, 1), lambda i: (i, 0)),
                  pl.BlockSpec(memory_space=pltpu.SMEM)],
        out_specs=mat,
        out_shape=jax.ShapeDtypeStruct((NP, H), jnp.float32),
    )(xp, a0, a1, b1, W1, W2, b2, score, t)


def _dec_body(xb_ref, a0_ref, a1_ref, b1_ref, w1_ref, w2_ref, b2_ref, wo_ref,
              bo_ref, o_ref):
    h = xb_ref[...] + a0_ref[...] + a1_ref[...]
    xd = _elu(_mm(_elu(_mm(h, w1_ref[...]) + b1_ref[...]), w2_ref[...])
              + b2_ref[...])
    o_ref[...] = _mm(xd, wo_ref[...]) + bo_ref[...]


def _decoder(xb, a0, a1, b1, W1, W2, b2, Wo, bo):
    NP, H = xb.shape
    C = Wo.shape[1]
    mat = pl.BlockSpec((_BLK, H), lambda i: (i, 0))
    vec = lambda: pl.BlockSpec((1, H), lambda i: (0, 0))
    sq = lambda: pl.BlockSpec((H, H), lambda i: (0, 0))
    return pl.pallas_call(
        _dec_body,
        grid=(NP // _BLK,),
        in_specs=[mat, mat, mat, vec(), sq(), sq(), vec(),
                  pl.BlockSpec((H, C), lambda i: (0, 0)),
                  pl.BlockSpec((1, C), lambda i: (0, 0))],
        out_specs=pl.BlockSpec((_BLK, C), lambda i: (i, 0)),
        out_shape=jax.ShapeDtypeStruct((NP, C), jnp.float32),
    )(xb, a0, a1, b1, W1, W2, b2, Wo, bo)


def _scatter_add_partials(y_pad, src3, dst3):
    NP, H = y_pad.shape
    CH = src3.shape[1]
    rows_per = NP // _NSUB
    mesh = plsc.VectorSubcoreMesh(core_axis_name="c", subcore_axis_name="s")

    @functools.partial(
        pl.kernel,
        out_type=jax.ShapeDtypeStruct((_NCORE, NP, H), jnp.float32),
        mesh=mesh,
        compiler_params=pltpu.CompilerParams(use_tc_tiling_on_sc=False),
        scratch_types=[
            pltpu.VMEM_SHARED((NP, H), jnp.float32),
            pltpu.VMEM((CH, 128), jnp.int32),
            pltpu.VMEM((CH, 128), jnp.int32),
            pltpu.VMEM((_CHUNK, H), jnp.float32),
            pltpu.VMEM((_ZROWS, H), jnp.float32),
            pltpu.SemaphoreType.DMA,
        ],
    )
    def run(y_hbm, src_hbm, dst_hbm, out_hbm, acc, src_v, dst_v, rows_v,
            zero_v, sem):
        cid = lax.axis_index("c")
        sid = lax.axis_index("s")
        wid = cid * _NSUB + sid

        pltpu.sync_copy(src_hbm.at[wid], src_v)
        pltpu.sync_copy(dst_hbm.at[wid], dst_v)

        z16 = jnp.zeros((16,), jnp.float32)

        def zbody(r, carry):
            for c in range(H // 16):
                zero_v[r, pl.ds(c * 16, 16)] = z16
            return carry

        lax.fori_loop(0, _ZROWS, zbody, 0)

        def zcopy(b, carry):
            pltpu.sync_copy(
                zero_v, acc.at[pl.ds(sid * rows_per + b * _ZROWS, _ZROWS)])
            return carry

        lax.fori_loop(0, rows_per // _ZROWS, zcopy, 0)
        plsc.subcore_barrier()

        def chunk(j, carry):
            pltpu.async_copy(y_hbm.at[src_v.at[j]], rows_v, sem).wait()
            pltpu.sync_copy(rows_v, acc.at[dst_v.at[j]], add=True)
            return carry

        lax.fori_loop(0, CH, chunk, 0)
        plsc.subcore_barrier()
        pltpu.sync_copy(acc.at[pl.ds(sid * rows_per, rows_per)],
                        out_hbm.at[cid, pl.ds(sid * rows_per, rows_per)])

    return run(y_pad, src3, dst3)


def kernel(x, edge_index, batch, W1e, b1e, gam, bet, W2e, b2e, w_pool,
           W1b, b1b, W2b, b2b, W1d, b1d, W2d, b2d, Wo, bo):
    N, D = x.shape
    E = edge_index.shape[1]
    K = -(-N // 2)

    NP = -(-(N + 1) // _BLK) * _BLK
    x_pad = jnp.pad(x, ((0, NP - N), (0, 0)))

    nw = _NCORE * _NSUB
    CH = -(-E // (nw * _CHUNK))
    tot = nw * CH * _CHUNK
    src = jnp.concatenate(
        [edge_index[0], jnp.full((tot - E,), N, jnp.int32)]).reshape(
            nw, CH, _CHUNK)
    dst = jnp.concatenate(
        [edge_index[1], jnp.full((tot - E,), N, jnp.int32)]).reshape(
            nw, CH, _CHUNK)

    r = lambda v: v.reshape(1, -1)

    ax = _scatter_add_partials(x_pad, src, dst)
    x1, score = _encoder(x_pad, ax[0], ax[1], r(b1e), r(gam), r(bet),
                         W1e, W2e, r(b2e), r(w_pool))

    t = _threshold(score.reshape(NP // 128, 128), N, K)

    xp = _pool(x1, score, t)
    ap = _scatter_add_partials(xp, src, dst)
    xb = _bottleneck(xp, ap[0], ap[1], r(b1b), W1b, W2b, r(b2b), score, t)

    ad = _scatter_add_partials(xb, src, dst)
    out = _decoder(xb, ad[0], ad[1], r(b1d), W1d, W2d, r(b2d), Wo, r(bo))
    return out[:N]
